# Initial kernel scaffold; baseline (speedup 1.0000x reference)
import jax
import jax.numpy as jnp
from jax import lax
from jax.experimental import pallas as pl
from jax.experimental.pallas import tpu as pltpu
from jax.experimental.pallas import tpu_sc as plsc

N = 10000
E = 320000
D = 128

NC = 2
NS = 16
NW = NC * NS
TPT = E // NW
C = 80
NCH = TPT // C
RPT = N // NS
ZR = 125


def _sc_body(ego_hbm, src_hbm, dst_hbm, ev_hbm, out_hbm,
             acc_sh, idx_v, dstidx_v, ev_v, rows_v, zero_v, sem):
    c = lax.axis_index("c")
    s = lax.axis_index("s")
    wid = s * NC + c

    def zfill(i, carry):
        for j in range(D // 16):
            zero_v[i, pl.ds(j * 16, 16)] = jnp.zeros((16,), jnp.float32)
        return carry
    lax.fori_loop(0, ZR, zfill, 0)
    row0 = s * RPT
    for k in range(RPT // ZR):
        pltpu.sync_copy(zero_v, acc_sh.at[pl.ds(row0 + k * ZR, ZR)])
    plsc.subcore_barrier()

    def chunk_body(g, carry):
        base = wid * TPT + g * C
        pltpu.sync_copy(src_hbm.at[pl.ds(base, C)], idx_v)
        pltpu.sync_copy(dst_hbm.at[pl.ds(base, C)], dstidx_v)
        pltpu.sync_copy(ev_hbm.at[pl.ds(base, C)], ev_v)
        pltpu.async_copy(ego_hbm.at[idx_v], rows_v, sem).wait()

        def row_body(e, carry2):
            sc = ev_v[e]
            for j in range(D // 16):
                rows_v[e, pl.ds(j * 16, 16)] = rows_v[e, pl.ds(j * 16, 16)] * sc
            return carry2
        lax.fori_loop(0, C, row_body, 0)

        pltpu.sync_copy(rows_v, acc_sh.at[dstidx_v], add=True)
        return carry
    lax.fori_loop(0, NCH, chunk_body, 0)

    plsc.subcore_barrier()

    pltpu.sync_copy(acc_sh.at[pl.ds(row0, RPT)],
                    out_hbm.at[pl.ds(c * N + row0, RPT)])


@jax.jit
def _sc_segment_sum(ego, src, dst, ev):
    mesh = plsc.VectorSubcoreMesh(core_axis_name="c", subcore_axis_name="s")
    f = pl.kernel(
        _sc_body,
        out_type=jax.ShapeDtypeStruct((NC * N, D), jnp.float32),
        mesh=mesh,
        scratch_types=[
            pltpu.VMEM_SHARED((N, D), jnp.float32),
            pltpu.VMEM((C,), jnp.int32),
            pltpu.VMEM((C,), jnp.int32),
            pltpu.VMEM((C,), jnp.float32),
            pltpu.VMEM((C, D), jnp.float32),
            pltpu.VMEM((ZR, D), jnp.float32),
            pltpu.SemaphoreType.DMA,
        ],
    )
    return f(ego, src, dst, ev)


def _tc_body(ego, p0, p1, w1t, b1, g1, be1, w2t, b2, g2, be2, out):
    e = ego[...]
    side = p0[...] + p1[...]

    def branch(x, wt, b, g, be):
        y = jnp.dot(x, wt[...], preferred_element_type=jnp.float32) + b[...]
        y = jnp.where(y >= 0, y, 0.01 * y)
        m = jnp.mean(y, axis=-1, keepdims=True)
        v = jnp.mean((y - m) ** 2, axis=-1, keepdims=True)
        return (y - m) * lax.rsqrt(v + 1e-5) * g[...] + be[...]

    out[...] = (branch(e + side, w1t, b1, g1, be1)
                + branch(e * side, w2t, b2, g2, be2))


R = 400


@jax.jit
def _tc_aggregate(ego, partials, W1, b1, W2, b2, g1, beta1, g2, beta2):
    p0 = partials[:N]
    p1 = partials[N:]
    w1t = W1.T
    w2t = W2.T
    row2 = lambda a: a.reshape(1, D)
    blk = pl.BlockSpec((R, D), lambda i: (i, 0))
    small = pl.BlockSpec((1, D), lambda i: (0, 0))
    wspec = pl.BlockSpec((D, D), lambda i: (0, 0))
    return pl.pallas_call(
        _tc_body,
        grid=(N // R,),
        in_specs=[blk, blk, blk, wspec, small, small, small,
                  wspec, small, small, small],
        out_specs=blk,
        out_shape=jax.ShapeDtypeStruct((N, D), jnp.float32),
    )(ego, p0, p1, w1t, row2(b1), row2(g1), row2(beta1),
      w2t, row2(b2), row2(g2), row2(beta2))


def kernel(ego_embeddings, edge_index, edge_values, W1, b1, W2, b2,
           g1, beta1, g2, beta2):
    dst = edge_index[0].astype(jnp.int32)
    src = edge_index[1].astype(jnp.int32)
    partials = _sc_segment_sum(ego_embeddings, src, dst, edge_values)
    return _tc_aggregate(ego_embeddings, partials, W1, b1, W2, b2,
                         g1, beta1, g2, beta2)


# speedup vs baseline: 4.3999x; 4.3999x over previous
import jax
import jax.numpy as jnp
from jax import lax
from jax.experimental import pallas as pl
from jax.experimental.pallas import tpu as pltpu
from jax.experimental.pallas import tpu_sc as plsc

N = 10000
E = 320000
D = 128

NC = 2
NS = 16
NW = NC * NS
TPT = E // NW
C = 80
NCH = TPT // C
NP = 10240
RPT = NP // NS
ZR = 128


def _sc_body(ego_hbm, src_hbm, dst_hbm, ev_hbm, out_hbm,
             acc_sh, idx_v, dstidx_v, ev_v, rows_v, zero_v, sem):
    c = lax.axis_index("c")
    s = lax.axis_index("s")
    wid = s * NC + c

    def zfill(i, carry):
        for j in range(D // 16):
            zero_v[i, pl.ds(j * 16, 16)] = jnp.zeros((16,), jnp.float32)
        return carry
    lax.fori_loop(0, ZR, zfill, 0)
    row0 = s * RPT
    for k in range(RPT // ZR):
        pltpu.sync_copy(zero_v, acc_sh.at[pl.ds(row0 + k * ZR, ZR)])
    plsc.subcore_barrier()

    def chunk_body(g, carry):
        base = wid * TPT + g * C
        pltpu.sync_copy(src_hbm.at[pl.ds(base, C)], idx_v)
        pltpu.sync_copy(dst_hbm.at[pl.ds(base, C)], dstidx_v)
        pltpu.sync_copy(ev_hbm.at[pl.ds(base, C)], ev_v)
        pltpu.async_copy(ego_hbm.at[idx_v], rows_v, sem).wait()

        def row_body(g2, carry2):
            e0 = g2 * 16
            evv = ev_v[pl.ds(e0, 16)]
            for k in range(16):
                sc = evv[k]
                for j in range(D // 16):
                    rows_v[e0 + k, pl.ds(j * 16, 16)] = (
                        rows_v[e0 + k, pl.ds(j * 16, 16)] * sc)
            return carry2
        lax.fori_loop(0, C // 16, row_body, 0)

        pltpu.sync_copy(rows_v, acc_sh.at[dstidx_v], add=True)
        return carry
    lax.fori_loop(0, NCH, chunk_body, 0)

    plsc.subcore_barrier()

    pltpu.sync_copy(acc_sh.at[pl.ds(row0, RPT)],
                    out_hbm.at[pl.ds(c * NP + row0, RPT)])


@jax.jit
def _sc_segment_sum(ego, src, dst, ev):
    mesh = plsc.VectorSubcoreMesh(core_axis_name="c", subcore_axis_name="s")
    f = pl.kernel(
        _sc_body,
        out_type=jax.ShapeDtypeStruct((NC * NP, D), jnp.float32),
        mesh=mesh,
        scratch_types=[
            pltpu.VMEM_SHARED((NP, D), jnp.float32),
            pltpu.VMEM((C,), jnp.int32),
            pltpu.VMEM((C,), jnp.int32),
            pltpu.VMEM((C,), jnp.float32),
            pltpu.VMEM((C, D), jnp.float32),
            pltpu.VMEM((ZR, D), jnp.float32),
            pltpu.SemaphoreType.DMA,
        ],
    )
    return f(ego, src, dst, ev)


def _tc_body(ego, p0, p1, w1t, b1, g1, be1, w2t, b2, g2, be2, out):
    e = ego[...]
    side = p0[...] + p1[...]

    def branch(x, wt, b, g, be):
        y = jnp.dot(x, wt[...], preferred_element_type=jnp.float32) + b[...]
        y = jnp.where(y >= 0, y, 0.01 * y)
        m = jnp.mean(y, axis=-1, keepdims=True)
        v = jnp.mean((y - m) ** 2, axis=-1, keepdims=True)
        return (y - m) * lax.rsqrt(v + 1e-5) * g[...] + be[...]

    out[...] = (branch(e + side, w1t, b1, g1, be1)
                + branch(e * side, w2t, b2, g2, be2))


R = 400


@jax.jit
def _tc_aggregate(ego, partials, W1, b1, W2, b2, g1, beta1, g2, beta2):
    p0 = partials[:N]
    p1 = partials[NP:NP + N]
    w1t = W1.T
    w2t = W2.T
    row2 = lambda a: a.reshape(1, D)
    blk = pl.BlockSpec((R, D), lambda i: (i, 0))
    small = pl.BlockSpec((1, D), lambda i: (0, 0))
    wspec = pl.BlockSpec((D, D), lambda i: (0, 0))
    return pl.pallas_call(
        _tc_body,
        grid=(N // R,),
        in_specs=[blk, blk, blk, wspec, small, small, small,
                  wspec, small, small, small],
        out_specs=blk,
        out_shape=jax.ShapeDtypeStruct((N, D), jnp.float32),
    )(ego, p0, p1, w1t, row2(b1), row2(g1), row2(beta1),
      w2t, row2(b2), row2(g2), row2(beta2))


def kernel(ego_embeddings, edge_index, edge_values, W1, b1, W2, b2,
           g1, beta1, g2, beta2):
    dst = edge_index[0].astype(jnp.int32)
    src = edge_index[1].astype(jnp.int32)
    partials = _sc_segment_sum(ego_embeddings, src, dst, edge_values)
    return _tc_aggregate(ego_embeddings, partials, W1, b1, W2, b2,
                         g1, beta1, g2, beta2)
